# baseline (speedup 1.0000x reference)
import functools

import jax
import jax.numpy as jnp
from jax import lax
from jax.experimental import pallas as pl
from jax.experimental.pallas import tpu as pltpu
from jax.experimental.pallas import tpu_sc as plsc

B, NR, P = 4, 2048, 64
D = H = W = 128
DHW = D * H * W
N = B * NR * P
NC, NS = 2, 16
NW = NC * NS
PPW = N // NW
CHK = 128
NCHUNK = PPW // CHK
NCORN = 8
LANES = 16
NGRP = CHK // LANES


def _interp_body(tab, px, py, pz, dens_out, feat_out,
                 px_v, py_v, pz_v, idx_v, w_v, rows_v, dens_v, feat_v, sem):
  wid = lax.axis_index("s") * NC + lax.axis_index("c")
  b = wid // (NW // B)
  row_base = b * DHW
  lane = lax.broadcasted_iota(jnp.int32, (LANES,), 0)

  def axis_prep(coord):
    f = (coord + 1.0) * ((D - 1) * 0.5)
    t = f.astype(jnp.int32)
    i0 = t - jnp.where(t.astype(jnp.float32) > f, 1, 0)
    w1 = f - i0.astype(jnp.float32)
    w0 = 1.0 - w1
    i1 = i0 + 1
    w0m = jnp.where((i0 >= 0) & (i0 < D), w0, 0.0)
    w1m = jnp.where((i1 >= 0) & (i1 < D), w1, 0.0)
    return jnp.clip(i0, 0, D - 1), jnp.clip(i1, 0, D - 1), w0m, w1m

  def chunk(k, carry):
    base = wid * PPW + k * CHK
    pltpu.sync_copy(px.at[pl.ds(base, CHK)], px_v)
    pltpu.sync_copy(py.at[pl.ds(base, CHK)], py_v)
    pltpu.sync_copy(pz.at[pl.ds(base, CHK)], pz_v)

    def build(g, c2):
      sl = pl.ds(g * LANES, LANES)
      x0, x1, wx0, wx1 = axis_prep(px_v[sl])
      y0, y1, wy0, wy1 = axis_prep(py_v[sl])
      z0, z1, wz0, wz1 = axis_prep(pz_v[sl])
      c = 0
      for zc, wz in ((z0, wz0), (z1, wz1)):
        for yc, wy in ((y0, wy0), (y1, wy1)):
          for xc, wx in ((x0, wx0), (x1, wx1)):
            idx_v[c, sl] = row_base + (zc * H + yc) * W + xc
            w_v[c, sl] = wz * wy * wx
            c += 1
      return c2
    lax.fori_loop(0, NGRP, build, 0)

    copies = [pltpu.async_copy(tab.at[idx_v.at[c]],
                               rows_v.at[pl.ds(c * CHK, CHK)], sem)
              for c in range(NCORN)]
    for cp in copies:
      cp.wait()

    def interp(g, c2):
      sl = pl.ds(g * LANES, LANES)
      p_vec = g * LANES + lane
      for ch in range(9):
        ch_vec = jnp.full((LANES,), ch, jnp.int32)
        acc = jnp.zeros((LANES,), jnp.float32)
        for c in range(NCORN):
          val = plsc.load_gather(rows_v, [p_vec + c * CHK, ch_vec])
          acc = acc + val * w_v[c, sl]
        if ch == 0:
          dens_v[sl] = acc
        else:
          plsc.store_scatter(feat_v, [(g * LANES + lane) * 8 + (ch - 1)], acc)
      return c2
    lax.fori_loop(0, NGRP, interp, 0)

    pltpu.sync_copy(dens_v, dens_out.at[pl.ds(base, CHK)])
    pltpu.sync_copy(feat_v, feat_out.at[pl.ds(base * 8, CHK * 8)])
    return carry

  lax.fori_loop(0, NCHUNK, chunk, 0)


def kernel(origins, directions, lengths, densities, features, world2local):
  ones = jnp.ones(origins.shape[:-1] + (1,), dtype=origins.dtype)
  o_h = jnp.concatenate([origins, ones], axis=-1)
  o_loc = jnp.einsum('bnk,bkj->bnj', o_h, world2local)
  o_loc = o_loc[..., :3] / o_loc[..., 3:4]
  d_loc = jnp.einsum('bnk,bkj->bnj', directions, world2local[:, :3, :3])
  pts = o_loc[:, :, None, :] + d_loc[:, :, None, :] * lengths[..., None]
  px = pts[..., 0].reshape(-1)
  py = pts[..., 1].reshape(-1)
  pz = pts[..., 2].reshape(-1)

  vol = jnp.concatenate([densities, features], axis=1).reshape(B, 9, DHW)
  eye = jnp.pad(jnp.eye(9, dtype=jnp.float32), ((0, 0), (0, 7)))
  tab = jnp.einsum('bcv,cd->bvd', vol, eye,
                   precision=jax.lax.Precision.HIGHEST).reshape(B * DHW, 16)

  mesh = plsc.VectorSubcoreMesh(core_axis_name="c", subcore_axis_name="s")
  run = pl.kernel(
      _interp_body,
      out_type=(jax.ShapeDtypeStruct((N,), jnp.float32),
                jax.ShapeDtypeStruct((N * 8,), jnp.float32)),
      mesh=mesh,
      scratch_types=(
          pltpu.VMEM((CHK,), jnp.float32),
          pltpu.VMEM((CHK,), jnp.float32),
          pltpu.VMEM((CHK,), jnp.float32),
          pltpu.VMEM((NCORN, CHK), jnp.int32),
          pltpu.VMEM((NCORN, CHK), jnp.float32),
          pltpu.VMEM((NCORN * CHK, 16), jnp.float32),
          pltpu.VMEM((CHK,), jnp.float32),
          pltpu.VMEM((CHK * 8,), jnp.float32),
          pltpu.SemaphoreType.DMA,
      ),
      compiler_params=pltpu.CompilerParams(needs_layout_passes=False,
                                           use_tc_tiling_on_sc=False),
  )
  dens_flat, feat_flat = run(tab, px, py, pz)
  return (dens_flat.reshape(B, NR, P, 1), feat_flat.reshape(B, NR, P, 8))


# speedup vs baseline: 5.7463x; 5.7463x over previous
import jax
import jax.numpy as jnp
from jax import lax
from jax.experimental import pallas as pl
from jax.experimental.pallas import tpu as pltpu
from jax.experimental.pallas import tpu_sc as plsc

B, NR, P = 4, 2048, 64
D = H = W = 128
DHW = D * H * W
N = B * NR * P
NC, NS = 2, 16
NW = NC * NS
PPW = N // NW
CHK = 128
NCHUNK = PPW // CHK
NCORN = 8
LANES = 16
NGRP = CHK // LANES
VPW = B * DHW // NW
TB = 2048
NBCHUNK = VPW // TB


def _body(dens, feat, px, py, pz, dens_out, feat_out, tab,
          strip_v, rows_v, px_v, py_v, pz_v, idx_v, w_v, corner_v,
          dens_v, feat_v, sem):
  wid = lax.axis_index("c") * NS + lax.axis_index("s")
  lane = lax.broadcasted_iota(jnp.int32, (LANES,), 0)

  slab = wid * VPW
  bb = slab // DHW
  lv0 = slab - bb * DHW

  def build_chunk(k, carry):
    lv = lv0 + k * TB
    cps = [pltpu.async_copy(
        dens.at[pl.ds(bb * DHW + lv, TB)], strip_v.at[0], sem)]
    cps += [pltpu.async_copy(
        feat.at[pl.ds((bb * 8 + ch) * DHW + lv, TB)], strip_v.at[ch + 1], sem)
        for ch in range(8)]
    for cp in cps:
      cp.wait()

    def interleave(g, c2):
      sl = pl.ds(g * LANES, LANES)
      row_idx = g * LANES + lane
      for ch in range(9):
        ch_vec = jnp.full((LANES,), ch, jnp.int32)
        plsc.store_scatter(rows_v, [row_idx, ch_vec], strip_v[ch, sl])
      return c2
    lax.fori_loop(0, TB // LANES, interleave, 0)
    pltpu.sync_copy(rows_v, tab.at[pl.ds(slab + k * TB, TB)])
    return carry

  lax.fori_loop(0, NBCHUNK, build_chunk, 0)
  plsc.subcore_barrier()

  def axis_prep(coord):
    f = (coord + 1.0) * ((D - 1) * 0.5)
    t = f.astype(jnp.int32)
    i0 = t - jnp.where(t.astype(jnp.float32) > f, 1, 0)
    w1 = f - i0.astype(jnp.float32)
    w0 = 1.0 - w1
    i1 = i0 + 1
    w0m = jnp.where((i0 >= 0) & (i0 < D), w0, 0.0)
    w1m = jnp.where((i1 >= 0) & (i1 < D), w1, 0.0)
    return jnp.clip(i0, 0, D - 1), jnp.clip(i1, 0, D - 1), w0m, w1m

  b = wid // (NW // B)
  row_base = b * DHW

  def chunk(k, carry):
    base = wid * PPW + k * CHK
    pltpu.sync_copy(px.at[pl.ds(base, CHK)], px_v)
    pltpu.sync_copy(py.at[pl.ds(base, CHK)], py_v)
    pltpu.sync_copy(pz.at[pl.ds(base, CHK)], pz_v)

    def build(g, c2):
      sl = pl.ds(g * LANES, LANES)
      x0, x1, wx0, wx1 = axis_prep(px_v[sl])
      y0, y1, wy0, wy1 = axis_prep(py_v[sl])
      z0, z1, wz0, wz1 = axis_prep(pz_v[sl])
      c = 0
      for zc, wz in ((z0, wz0), (z1, wz1)):
        for yc, wy in ((y0, wy0), (y1, wy1)):
          for xc, wx in ((x0, wx0), (x1, wx1)):
            idx_v[c, sl] = row_base + (zc * H + yc) * W + xc
            w_v[c, sl] = wz * wy * wx
            c += 1
      return c2
    lax.fori_loop(0, NGRP, build, 0)

    copies = [pltpu.async_copy(tab.at[idx_v.at[c]],
                               corner_v.at[pl.ds(c * CHK, CHK)], sem)
              for c in range(NCORN)]
    for cp in copies:
      cp.wait()

    def interp(g, c2):
      sl = pl.ds(g * LANES, LANES)
      p_vec = g * LANES + lane
      for ch in range(9):
        ch_vec = jnp.full((LANES,), ch, jnp.int32)
        acc = jnp.zeros((LANES,), jnp.float32)
        for c in range(NCORN):
          val = plsc.load_gather(corner_v, [p_vec + c * CHK, ch_vec])
          acc = acc + val * w_v[c, sl]
        if ch == 0:
          dens_v[sl] = acc
        else:
          plsc.store_scatter(feat_v, [(g * LANES + lane) * 8 + (ch - 1)], acc)
      return c2
    lax.fori_loop(0, NGRP, interp, 0)

    pltpu.sync_copy(dens_v, dens_out.at[pl.ds(base, CHK)])
    pltpu.sync_copy(feat_v, feat_out.at[pl.ds(base * 8, CHK * 8)])
    return carry

  lax.fori_loop(0, NCHUNK, chunk, 0)


def kernel(origins, directions, lengths, densities, features, world2local):
  ones = jnp.ones(origins.shape[:-1] + (1,), dtype=origins.dtype)
  o_h = jnp.concatenate([origins, ones], axis=-1)
  o_loc = jnp.einsum('bnk,bkj->bnj', o_h, world2local)
  o_loc = o_loc[..., :3] / o_loc[..., 3:4]
  d_loc = jnp.einsum('bnk,bkj->bnj', directions, world2local[:, :3, :3])
  pts = o_loc[:, :, None, :] + d_loc[:, :, None, :] * lengths[..., None]
  px = pts[..., 0].reshape(-1)
  py = pts[..., 1].reshape(-1)
  pz = pts[..., 2].reshape(-1)

  mesh = plsc.VectorSubcoreMesh(core_axis_name="c", subcore_axis_name="s")
  run = pl.kernel(
      _body,
      out_type=(jax.ShapeDtypeStruct((N,), jnp.float32),
                jax.ShapeDtypeStruct((N * 8,), jnp.float32),
                jax.ShapeDtypeStruct((B * DHW, 16), jnp.float32)),
      mesh=mesh,
      scratch_types=(
          pltpu.VMEM((9, TB), jnp.float32),
          pltpu.VMEM((TB, 16), jnp.float32),
          pltpu.VMEM((CHK,), jnp.float32),
          pltpu.VMEM((CHK,), jnp.float32),
          pltpu.VMEM((CHK,), jnp.float32),
          pltpu.VMEM((NCORN, CHK), jnp.int32),
          pltpu.VMEM((NCORN, CHK), jnp.float32),
          pltpu.VMEM((NCORN * CHK, 16), jnp.float32),
          pltpu.VMEM((CHK,), jnp.float32),
          pltpu.VMEM((CHK * 8,), jnp.float32),
          pltpu.SemaphoreType.DMA,
      ),
      compiler_params=pltpu.CompilerParams(needs_layout_passes=False,
                                           use_tc_tiling_on_sc=False),
  )
  dens_flat, feat_flat, _ = run(densities.reshape(-1), features.reshape(-1),
                                px, py, pz)
  return (dens_flat.reshape(B, NR, P, 1), feat_flat.reshape(B, NR, P, 8))


# speedup vs baseline: 9.0633x; 1.5772x over previous
import jax
import jax.numpy as jnp
from jax import lax
from jax.experimental import pallas as pl
from jax.experimental.pallas import tpu as pltpu
from jax.experimental.pallas import tpu_sc as plsc

B, NR, P = 4, 2048, 64
D = H = W = 128
DHW = D * H * W
N = B * NR * P
NC, NS = 2, 16
NW = NC * NS
PPW = N // NW
CHK = 128
NCHUNK = PPW // CHK
NCORN = 8
LANES = 16
NGRP = CHK // LANES
VPW = B * DHW // NW
TB = 1024
NBCHUNK = VPW // TB


def _body(dens, feat, px, py, pz, dens_out, feat_out, tab, dummy, dummy2,
          strip_v, rows_v, px_v, py_v, pz_v, idx_v, w_v, corner_v,
          dens_v, feat_v, sem_s, sem_w, sem_p, sem_g, sem_o):
  wid = lax.axis_index("c") * NS + lax.axis_index("s")
  lane = lax.broadcasted_iota(jnp.int32, (LANES,), 0)

  pbase0 = wid * PPW
  pltpu.async_copy(px.at[pl.ds(pbase0, CHK)], px_v.at[0, 0], sem_p.at[0])
  pltpu.async_copy(py.at[pl.ds(pbase0, CHK)], py_v.at[0, 0], sem_p.at[0])
  pltpu.async_copy(pz.at[pl.ds(pbase0, CHK)], pz_v.at[0, 0], sem_p.at[0])

  slab = wid * VPW
  bb = slab // DHW
  lv0 = slab - bb * DHW

  def strips_issue(k, s):
    lv = lv0 + k * TB
    pltpu.async_copy(dens.at[pl.ds(bb * DHW + lv, TB)],
                     strip_v.at[s, 0], sem_s.at[s])
    for ch in range(8):
      pltpu.async_copy(feat.at[pl.ds((bb * 8 + ch) * DHW + lv, TB)],
                       strip_v.at[s, ch + 1], sem_s.at[s])

  def strips_wait(s):
    for ch in range(9):
      pltpu.make_async_copy(dens.at[pl.ds(bb * DHW, TB)],
                            strip_v.at[s, ch], sem_s.at[s]).wait()

  def build_step(k, s):
    strips_wait(s)
    strips_issue(jnp.minimum(k + 1, NBCHUNK - 1), 1 - s)

    def interleave(g, c2):
      sl = pl.ds(g * LANES, LANES)
      row_idx = s * TB + g * LANES + lane
      for ch in range(9):
        ch_vec = jnp.full((LANES,), ch, jnp.int32)
        plsc.store_scatter(rows_v, [row_idx, ch_vec], strip_v[s, ch, sl])
      return c2
    pltpu.make_async_copy(rows_v.at[pl.ds(s * TB, TB)],
                          dummy.at[pl.ds(0, TB)], sem_w.at[s]).wait()
    lax.fori_loop(0, TB // LANES, interleave, 0)
    pltpu.async_copy(rows_v.at[pl.ds(s * TB, TB)],
                     tab.at[pl.ds(slab + k * TB, TB)], sem_w.at[s])

  for s in (0, 1):
    pltpu.async_copy(rows_v.at[pl.ds(s * TB, TB)],
                     dummy.at[pl.ds(0, TB)], sem_w.at[s])
  strips_issue(0, 0)

  def build_pair(ko, carry):
    build_step(2 * ko, 0)
    build_step(2 * ko + 1, 1)
    return carry
  lax.fori_loop(0, NBCHUNK // 2, build_pair, 0)

  strips_wait(0)
  for s in (0, 1):
    pltpu.make_async_copy(rows_v.at[pl.ds(s * TB, TB)],
                          tab.at[pl.ds(slab, TB)], sem_w.at[s]).wait()
  plsc.subcore_barrier()

  def axis_prep(coord):
    f = (coord + 1.0) * ((D - 1) * 0.5)
    t = f.astype(jnp.int32)
    i0 = t - jnp.where(t.astype(jnp.float32) > f, 1, 0)
    w1 = f - i0.astype(jnp.float32)
    w0 = 1.0 - w1
    i1 = i0 + 1
    w0m = jnp.where((i0 >= 0) & (i0 < D), w0, 0.0)
    w1m = jnp.where((i1 >= 0) & (i1 < D), w1, 0.0)
    return jnp.clip(i0, 0, D - 1), jnp.clip(i1, 0, D - 1), w0m, w1m

  b = wid // (NW // B)
  row_base = b * DHW

  def stage_a(k, s):
    for pv in (px_v, py_v, pz_v):
      pltpu.make_async_copy(px.at[pl.ds(pbase0, CHK)],
                            pv.at[s, 0], sem_p.at[s]).wait()

    def build(g, c2):
      sl = pl.ds(g * LANES, LANES)
      x0, x1, wx0, wx1 = axis_prep(px_v[s, 0, sl])
      y0, y1, wy0, wy1 = axis_prep(py_v[s, 0, sl])
      z0, z1, wz0, wz1 = axis_prep(pz_v[s, 0, sl])
      c = 0
      for zc, wz in ((z0, wz0), (z1, wz1)):
        for yc, wy in ((y0, wy0), (y1, wy1)):
          for xc, wx in ((x0, wx0), (x1, wx1)):
            idx_v[s, c, sl] = row_base + (zc * H + yc) * W + xc
            w_v[s, c, sl] = wz * wy * wx
            c += 1
      return c2
    lax.fori_loop(0, NGRP, build, 0)

    for c in range(NCORN):
      pltpu.async_copy(tab.at[idx_v.at[s, c]],
                       corner_v.at[pl.ds((s * NCORN + c) * CHK, CHK)],
                       sem_g.at[s])
    nbase = wid * PPW + jnp.minimum(k + 1, NCHUNK - 1) * CHK
    pltpu.async_copy(px.at[pl.ds(nbase, CHK)], px_v.at[1 - s, 0], sem_p.at[1 - s])
    pltpu.async_copy(py.at[pl.ds(nbase, CHK)], py_v.at[1 - s, 0], sem_p.at[1 - s])
    pltpu.async_copy(pz.at[pl.ds(nbase, CHK)], pz_v.at[1 - s, 0], sem_p.at[1 - s])

  def stage_b(k, s):
    for c in range(NCORN):
      pltpu.make_async_copy(tab.at[idx_v.at[s, c]],
                            corner_v.at[pl.ds((s * NCORN + c) * CHK, CHK)],
                            sem_g.at[s]).wait()
    pltpu.make_async_copy(dens_v.at[s], dummy2.at[pl.ds(0, CHK)],
                          sem_o.at[s]).wait()
    pltpu.make_async_copy(feat_v.at[pl.ds(s * CHK * 8, CHK * 8)],
                          dummy2.at[pl.ds(0, CHK * 8)], sem_o.at[s]).wait()

    def interp(g, c2):
      sl = pl.ds(g * LANES, LANES)
      p_vec = g * LANES + lane
      for ch in range(9):
        ch_vec = jnp.full((LANES,), ch, jnp.int32)
        acc = jnp.zeros((LANES,), jnp.float32)
        for c in range(NCORN):
          val = plsc.load_gather(corner_v,
                                 [p_vec + (s * NCORN + c) * CHK, ch_vec])
          acc = acc + val * w_v[s, c, sl]
        if ch == 0:
          dens_v[s, sl] = acc
        else:
          plsc.store_scatter(feat_v,
                             [s * CHK * 8 + (g * LANES + lane) * 8 + (ch - 1)],
                             acc)
      return c2
    lax.fori_loop(0, NGRP, interp, 0)
    base = wid * PPW + k * CHK
    pltpu.async_copy(dens_v.at[s], dens_out.at[pl.ds(base, CHK)], sem_o.at[s])
    pltpu.async_copy(feat_v.at[pl.ds(s * CHK * 8, CHK * 8)],
                     feat_out.at[pl.ds(base * 8, CHK * 8)], sem_o.at[s])

  for s in (0, 1):
    pltpu.async_copy(dens_v.at[s], dummy2.at[pl.ds(0, CHK)], sem_o.at[s])
    pltpu.async_copy(feat_v.at[pl.ds(s * CHK * 8, CHK * 8)],
                     dummy2.at[pl.ds(0, CHK * 8)], sem_o.at[s])

  stage_a(0, 0)
  stage_a(1, 1)
  stage_b(0, 0)

  def sample_pair(ko, carry):
    stage_a(2 * ko + 2, 0)
    stage_b(2 * ko + 1, 1)
    stage_a(2 * ko + 3, 1)
    stage_b(2 * ko + 2, 0)
    return carry
  lax.fori_loop(0, (NCHUNK - 2) // 2, sample_pair, 0)
  stage_b(NCHUNK - 1, 1)

  for pv in (px_v, py_v, pz_v):
    pltpu.make_async_copy(px.at[pl.ds(pbase0, CHK)],
                          pv.at[0, 0], sem_p.at[0]).wait()
  for s in (0, 1):
    pltpu.make_async_copy(dens_v.at[s], dummy2.at[pl.ds(0, CHK)],
                          sem_o.at[s]).wait()
    pltpu.make_async_copy(feat_v.at[pl.ds(s * CHK * 8, CHK * 8)],
                          dummy2.at[pl.ds(0, CHK * 8)], sem_o.at[s]).wait()


def kernel(origins, directions, lengths, densities, features, world2local):
  ones = jnp.ones(origins.shape[:-1] + (1,), dtype=origins.dtype)
  o_h = jnp.concatenate([origins, ones], axis=-1)
  o_loc = jnp.einsum('bnk,bkj->bnj', o_h, world2local)
  o_loc = o_loc[..., :3] / o_loc[..., 3:4]
  d_loc = jnp.einsum('bnk,bkj->bnj', directions, world2local[:, :3, :3])
  pts = o_loc[:, :, None, :] + d_loc[:, :, None, :] * lengths[..., None]
  px = pts[..., 0].reshape(-1)
  py = pts[..., 1].reshape(-1)
  pz = pts[..., 2].reshape(-1)

  mesh = plsc.VectorSubcoreMesh(core_axis_name="c", subcore_axis_name="s")
  run = pl.kernel(
      _body,
      out_type=(jax.ShapeDtypeStruct((N,), jnp.float32),
                jax.ShapeDtypeStruct((N * 8,), jnp.float32),
                jax.ShapeDtypeStruct((B * DHW, 16), jnp.float32),
                jax.ShapeDtypeStruct((TB, 16), jnp.float32),
                jax.ShapeDtypeStruct((CHK * 8,), jnp.float32)),
      mesh=mesh,
      scratch_types=(
          pltpu.VMEM((2, 9, TB), jnp.float32),
          pltpu.VMEM((2 * TB, 16), jnp.float32),
          pltpu.VMEM((2, 1, CHK), jnp.float32),
          pltpu.VMEM((2, 1, CHK), jnp.float32),
          pltpu.VMEM((2, 1, CHK), jnp.float32),
          pltpu.VMEM((2, NCORN, CHK), jnp.int32),
          pltpu.VMEM((2, NCORN, CHK), jnp.float32),
          pltpu.VMEM((2 * NCORN * CHK, 16), jnp.float32),
          pltpu.VMEM((2, CHK), jnp.float32),
          pltpu.VMEM((2 * CHK * 8,), jnp.float32),
          pltpu.SemaphoreType.DMA((2,)),
          pltpu.SemaphoreType.DMA((2,)),
          pltpu.SemaphoreType.DMA((2,)),
          pltpu.SemaphoreType.DMA((2,)),
          pltpu.SemaphoreType.DMA((2,)),
      ),
      compiler_params=pltpu.CompilerParams(needs_layout_passes=False,
                                           use_tc_tiling_on_sc=False),
  )
  dens_flat, feat_flat, _, _, _ = run(densities.reshape(-1),
                                   features.reshape(-1), px, py, pz)
  return (dens_flat.reshape(B, NR, P, 1), feat_flat.reshape(B, NR, P, 8))


# speedup vs baseline: 13.7910x; 1.5216x over previous
import jax
import jax.numpy as jnp
from jax import lax
from jax.experimental import pallas as pl
from jax.experimental.pallas import tpu as pltpu
from jax.experimental.pallas import tpu_sc as plsc

B, NR, P = 4, 2048, 64
D = H = W = 128
DHW = D * H * W
N = B * NR * P
NC, NS = 2, 16
NW = NC * NS
PPW = N // NW
CHK = 128
NCHUNK = PPW // CHK
NCORN = 8
LANES = 16
NGRP = CHK // LANES
VPW = B * DHW // NW
TB = 1024
NBCHUNK = VPW // TB


def _body(dens, feat, px, py, pz, dens_out, feat_out, tab, dummy, dummy2,
          strip_v, rows_v, px_v, py_v, pz_v, idx_v, w_v, corner_v,
          dens_v, feat_v, sem_s, sem_w, sem_p, sem_g, sem_o):
  wid = lax.axis_index("c") * NS + lax.axis_index("s")
  lane = lax.broadcasted_iota(jnp.int32, (LANES,), 0)

  pbase0 = wid * PPW
  pltpu.async_copy(px.at[pl.ds(pbase0, CHK)], px_v.at[0, 0], sem_p.at[0])
  pltpu.async_copy(py.at[pl.ds(pbase0, CHK)], py_v.at[0, 0], sem_p.at[0])
  pltpu.async_copy(pz.at[pl.ds(pbase0, CHK)], pz_v.at[0, 0], sem_p.at[0])

  slab = wid * VPW
  bb = slab // DHW
  lv0 = slab - bb * DHW

  def strips_issue(k, s):
    lv = lv0 + k * TB
    pltpu.async_copy(dens.at[pl.ds(bb * DHW + lv, TB)],
                     strip_v.at[s, 0], sem_s.at[s])
    for ch in range(8):
      pltpu.async_copy(feat.at[pl.ds((bb * 8 + ch) * DHW + lv, TB)],
                       strip_v.at[s, ch + 1], sem_s.at[s])

  def strips_wait(s):
    for ch in range(9):
      pltpu.make_async_copy(dens.at[pl.ds(bb * DHW, TB)],
                            strip_v.at[s, ch], sem_s.at[s]).wait()

  def build_step(k, s):
    strips_wait(s)
    strips_issue(jnp.minimum(k + 1, NBCHUNK - 1), 1 - s)

    zero16 = jnp.zeros((LANES,), jnp.float32)

    def interleave(g, c2):
      sl = pl.ds(g * LANES, LANES)
      row_idx = s * TB + g * LANES + lane
      for w in range(5):
        va = strip_v[s, 2 * w, sl]
        vb = strip_v[s, 2 * w + 1, sl] if w < 4 else zero16
        packed = plsc.pack(va, vb, format=plsc.PackFormat.INTERLEAVED)
        word = plsc.bitcast(packed, jnp.int32)
        w_vec = jnp.full((LANES,), w, jnp.int32)
        plsc.store_scatter(rows_v, [row_idx, w_vec], word)
      return c2
    pltpu.make_async_copy(rows_v.at[pl.ds(s * TB, TB)],
                          dummy.at[pl.ds(0, TB)], sem_w.at[s]).wait()
    lax.fori_loop(0, TB // LANES, interleave, 0)
    pltpu.async_copy(rows_v.at[pl.ds(s * TB, TB)],
                     tab.at[pl.ds(slab + k * TB, TB)], sem_w.at[s])

  for s in (0, 1):
    pltpu.async_copy(rows_v.at[pl.ds(s * TB, TB)],
                     dummy.at[pl.ds(0, TB)], sem_w.at[s])
  strips_issue(0, 0)

  def build_pair(ko, carry):
    build_step(2 * ko, 0)
    build_step(2 * ko + 1, 1)
    return carry
  lax.fori_loop(0, NBCHUNK // 2, build_pair, 0)

  strips_wait(0)
  for s in (0, 1):
    pltpu.make_async_copy(rows_v.at[pl.ds(s * TB, TB)],
                          tab.at[pl.ds(slab, TB)], sem_w.at[s]).wait()
  plsc.subcore_barrier()

  def axis_prep(coord):
    f = (coord + 1.0) * ((D - 1) * 0.5)
    t = f.astype(jnp.int32)
    i0 = t - jnp.where(t.astype(jnp.float32) > f, 1, 0)
    w1 = f - i0.astype(jnp.float32)
    w0 = 1.0 - w1
    i1 = i0 + 1
    w0m = jnp.where((i0 >= 0) & (i0 < D), w0, 0.0)
    w1m = jnp.where((i1 >= 0) & (i1 < D), w1, 0.0)
    return jnp.clip(i0, 0, D - 1), jnp.clip(i1, 0, D - 1), w0m, w1m

  b = wid // (NW // B)
  row_base = b * DHW

  def stage_a(k, s):
    for pv in (px_v, py_v, pz_v):
      pltpu.make_async_copy(px.at[pl.ds(pbase0, CHK)],
                            pv.at[s, 0], sem_p.at[s]).wait()

    def build(g, c2):
      sl = pl.ds(g * LANES, LANES)
      x0, x1, wx0, wx1 = axis_prep(px_v[s, 0, sl])
      y0, y1, wy0, wy1 = axis_prep(py_v[s, 0, sl])
      z0, z1, wz0, wz1 = axis_prep(pz_v[s, 0, sl])
      c = 0
      for zc, wz in ((z0, wz0), (z1, wz1)):
        for yc, wy in ((y0, wy0), (y1, wy1)):
          for xc, wx in ((x0, wx0), (x1, wx1)):
            idx_v[s, c, sl] = row_base + (zc * H + yc) * W + xc
            w_v[s, c, sl] = wz * wy * wx
            c += 1
      return c2
    lax.fori_loop(0, NGRP, build, 0)

    for c in range(NCORN):
      pltpu.async_copy(tab.at[idx_v.at[s, c]],
                       corner_v.at[pl.ds((s * NCORN + c) * CHK, CHK)],
                       sem_g.at[s])
    nbase = wid * PPW + jnp.minimum(k + 1, NCHUNK - 1) * CHK
    pltpu.async_copy(px.at[pl.ds(nbase, CHK)], px_v.at[1 - s, 0], sem_p.at[1 - s])
    pltpu.async_copy(py.at[pl.ds(nbase, CHK)], py_v.at[1 - s, 0], sem_p.at[1 - s])
    pltpu.async_copy(pz.at[pl.ds(nbase, CHK)], pz_v.at[1 - s, 0], sem_p.at[1 - s])

  def stage_b(k, s):
    for c in range(NCORN):
      pltpu.make_async_copy(tab.at[idx_v.at[s, c]],
                            corner_v.at[pl.ds((s * NCORN + c) * CHK, CHK)],
                            sem_g.at[s]).wait()
    pltpu.make_async_copy(dens_v.at[s], dummy2.at[pl.ds(0, CHK)],
                          sem_o.at[s]).wait()
    pltpu.make_async_copy(feat_v.at[pl.ds(s * CHK * 8, CHK * 8)],
                          dummy2.at[pl.ds(0, CHK * 8)], sem_o.at[s]).wait()

    def interp(g, c2):
      sl = pl.ds(g * LANES, LANES)
      p_vec = g * LANES + lane
      accs = [jnp.zeros((LANES,), jnp.float32) for _ in range(9)]
      for c in range(NCORN):
        wv = w_v[s, c, sl]
        for w in range(5):
          w_vec = jnp.full((LANES,), w, jnp.int32)
          word = plsc.load_gather(corner_v,
                                  [p_vec + (s * NCORN + c) * CHK, w_vec])
          packed = plsc.bitcast(word, jnp.bfloat16)
          va, vb = plsc.unpack(packed, format=plsc.PackFormat.INTERLEAVED)
          accs[2 * w] = accs[2 * w] + va.astype(jnp.float32) * wv
          if w < 4:
            accs[2 * w + 1] = accs[2 * w + 1] + vb.astype(jnp.float32) * wv
      dens_v[s, sl] = accs[0]
      for ch in range(1, 9):
        plsc.store_scatter(feat_v,
                           [s * CHK * 8 + (g * LANES + lane) * 8 + (ch - 1)],
                           accs[ch])
      return c2
    lax.fori_loop(0, NGRP, interp, 0)
    base = wid * PPW + k * CHK
    pltpu.async_copy(dens_v.at[s], dens_out.at[pl.ds(base, CHK)], sem_o.at[s])
    pltpu.async_copy(feat_v.at[pl.ds(s * CHK * 8, CHK * 8)],
                     feat_out.at[pl.ds(base * 8, CHK * 8)], sem_o.at[s])

  for s in (0, 1):
    pltpu.async_copy(dens_v.at[s], dummy2.at[pl.ds(0, CHK)], sem_o.at[s])
    pltpu.async_copy(feat_v.at[pl.ds(s * CHK * 8, CHK * 8)],
                     dummy2.at[pl.ds(0, CHK * 8)], sem_o.at[s])

  stage_a(0, 0)
  stage_a(1, 1)
  stage_b(0, 0)

  def sample_pair(ko, carry):
    stage_a(2 * ko + 2, 0)
    stage_b(2 * ko + 1, 1)
    stage_a(2 * ko + 3, 1)
    stage_b(2 * ko + 2, 0)
    return carry
  lax.fori_loop(0, (NCHUNK - 2) // 2, sample_pair, 0)
  stage_b(NCHUNK - 1, 1)

  for pv in (px_v, py_v, pz_v):
    pltpu.make_async_copy(px.at[pl.ds(pbase0, CHK)],
                          pv.at[0, 0], sem_p.at[0]).wait()
  for s in (0, 1):
    pltpu.make_async_copy(dens_v.at[s], dummy2.at[pl.ds(0, CHK)],
                          sem_o.at[s]).wait()
    pltpu.make_async_copy(feat_v.at[pl.ds(s * CHK * 8, CHK * 8)],
                          dummy2.at[pl.ds(0, CHK * 8)], sem_o.at[s]).wait()


def kernel(origins, directions, lengths, densities, features, world2local):
  ones = jnp.ones(origins.shape[:-1] + (1,), dtype=origins.dtype)
  o_h = jnp.concatenate([origins, ones], axis=-1)
  o_loc = jnp.einsum('bnk,bkj->bnj', o_h, world2local)
  o_loc = o_loc[..., :3] / o_loc[..., 3:4]
  d_loc = jnp.einsum('bnk,bkj->bnj', directions, world2local[:, :3, :3])
  pts = o_loc[:, :, None, :] + d_loc[:, :, None, :] * lengths[..., None]
  px = pts[..., 0].reshape(-1)
  py = pts[..., 1].reshape(-1)
  pz = pts[..., 2].reshape(-1)

  mesh = plsc.VectorSubcoreMesh(core_axis_name="c", subcore_axis_name="s")
  run = pl.kernel(
      _body,
      out_type=(jax.ShapeDtypeStruct((N,), jnp.float32),
                jax.ShapeDtypeStruct((N * 8,), jnp.float32),
                jax.ShapeDtypeStruct((B * DHW, 8), jnp.int32),
                jax.ShapeDtypeStruct((TB, 8), jnp.int32),
                jax.ShapeDtypeStruct((CHK * 8,), jnp.float32)),
      mesh=mesh,
      scratch_types=(
          pltpu.VMEM((2, 9, TB), jnp.float32),
          pltpu.VMEM((2 * TB, 8), jnp.int32),
          pltpu.VMEM((2, 1, CHK), jnp.float32),
          pltpu.VMEM((2, 1, CHK), jnp.float32),
          pltpu.VMEM((2, 1, CHK), jnp.float32),
          pltpu.VMEM((2, NCORN, CHK), jnp.int32),
          pltpu.VMEM((2, NCORN, CHK), jnp.float32),
          pltpu.VMEM((2 * NCORN * CHK, 8), jnp.int32),
          pltpu.VMEM((2, CHK), jnp.float32),
          pltpu.VMEM((2 * CHK * 8,), jnp.float32),
          pltpu.SemaphoreType.DMA((2,)),
          pltpu.SemaphoreType.DMA((2,)),
          pltpu.SemaphoreType.DMA((2,)),
          pltpu.SemaphoreType.DMA((2,)),
          pltpu.SemaphoreType.DMA((2,)),
      ),
      compiler_params=pltpu.CompilerParams(needs_layout_passes=False,
                                           use_tc_tiling_on_sc=False),
  )
  dens_flat, feat_flat, _, _, _ = run(densities.reshape(-1),
                                   features.reshape(-1), px, py, pz)
  return (dens_flat.reshape(B, NR, P, 1), feat_flat.reshape(B, NR, P, 8))


# speedup vs baseline: 14.0137x; 1.0162x over previous
import jax
import jax.numpy as jnp
from jax import lax
from jax.experimental import pallas as pl
from jax.experimental.pallas import tpu as pltpu
from jax.experimental.pallas import tpu_sc as plsc

B, NR, P = 4, 2048, 64
D = H = W = 128
DHW = D * H * W
N = B * NR * P
NC, NS = 2, 16
NW = NC * NS
PPW = N // NW
CHK = 128
NCHUNK = PPW // CHK
NCORN = 8
LANES = 16
NGRP = CHK // LANES
VPW = B * DHW // NW
TB = 1024
NBCHUNK = VPW // TB


def _body(dens, feat, oall, dall, tlen, dens_out, feat_out, tab, dummy, dummy2,
          strip_v, rows_v, oall_v, dall_v, len_v, idx_v, w_v, corner_v,
          dens_v, feat_v, sem_s, sem_w, sem_p, sem_g, sem_o):
  wid = lax.axis_index("c") * NS + lax.axis_index("s")
  lane = lax.broadcasted_iota(jnp.int32, (LANES,), 0)

  pbase0 = wid * PPW
  RPW = PPW // P
  pltpu.sync_copy(oall.at[pl.ds(wid * RPW * 3, RPW * 3)], oall_v)
  pltpu.sync_copy(dall.at[pl.ds(wid * RPW * 3, RPW * 3)], dall_v)
  pltpu.async_copy(tlen.at[pl.ds(pbase0, CHK)], len_v.at[0, 0], sem_p.at[0])

  slab = wid * VPW
  bb = slab // DHW
  lv0 = slab - bb * DHW

  def strips_issue(k, s):
    lv = lv0 + k * TB
    pltpu.async_copy(dens.at[pl.ds(bb * DHW + lv, TB)],
                     strip_v.at[s, 0], sem_s.at[s])
    for ch in range(8):
      pltpu.async_copy(feat.at[pl.ds((bb * 8 + ch) * DHW + lv, TB)],
                       strip_v.at[s, ch + 1], sem_s.at[s])

  def strips_wait(s):
    for ch in range(9):
      pltpu.make_async_copy(dens.at[pl.ds(bb * DHW, TB)],
                            strip_v.at[s, ch], sem_s.at[s]).wait()

  def build_step(k, s):
    strips_wait(s)
    strips_issue(jnp.minimum(k + 1, NBCHUNK - 1), 1 - s)

    zero16 = jnp.zeros((LANES,), jnp.float32)

    def interleave(g, c2):
      sl = pl.ds(g * LANES, LANES)
      row_idx = s * TB + g * LANES + lane
      for w in range(5):
        va = strip_v[s, 2 * w, sl]
        vb = strip_v[s, 2 * w + 1, sl] if w < 4 else zero16
        packed = plsc.pack(va, vb, format=plsc.PackFormat.INTERLEAVED)
        word = plsc.bitcast(packed, jnp.int32)
        w_vec = jnp.full((LANES,), w, jnp.int32)
        plsc.store_scatter(rows_v, [row_idx, w_vec], word)
      return c2
    pltpu.make_async_copy(rows_v.at[pl.ds(s * TB, TB)],
                          dummy.at[pl.ds(0, TB)], sem_w.at[s]).wait()
    lax.fori_loop(0, TB // LANES, interleave, 0)
    pltpu.async_copy(rows_v.at[pl.ds(s * TB, TB)],
                     tab.at[pl.ds(slab + k * TB, TB)], sem_w.at[s])

  for s in (0, 1):
    pltpu.async_copy(rows_v.at[pl.ds(s * TB, TB)],
                     dummy.at[pl.ds(0, TB)], sem_w.at[s])
  strips_issue(0, 0)

  def build_pair(ko, carry):
    build_step(2 * ko, 0)
    build_step(2 * ko + 1, 1)
    return carry
  lax.fori_loop(0, NBCHUNK // 2, build_pair, 0)

  strips_wait(0)
  for s in (0, 1):
    pltpu.make_async_copy(rows_v.at[pl.ds(s * TB, TB)],
                          tab.at[pl.ds(slab, TB)], sem_w.at[s]).wait()
  plsc.subcore_barrier()

  def axis_prep(coord):
    f = (coord + 1.0) * ((D - 1) * 0.5)
    t = f.astype(jnp.int32)
    i0 = t - jnp.where(t.astype(jnp.float32) > f, 1, 0)
    w1 = f - i0.astype(jnp.float32)
    w0 = 1.0 - w1
    i1 = i0 + 1
    w0m = jnp.where((i0 >= 0) & (i0 < D), w0, 0.0)
    w1m = jnp.where((i1 >= 0) & (i1 < D), w1, 0.0)
    return jnp.clip(i0, 0, D - 1), jnp.clip(i1, 0, D - 1), w0m, w1m

  b = wid // (NW // B)
  row_base = b * DHW

  def stage_a(k, s):
    pltpu.make_async_copy(tlen.at[pl.ds(pbase0, CHK)],
                          len_v.at[s, 0], sem_p.at[s]).wait()

    def build(g, c2):
      sl = pl.ds(g * LANES, LANES)
      t = len_v[s, 0, sl]
      ray = k * (CHK // P) + g // (P // LANES)
      def splat(ref, i):
        return plsc.load_gather(ref, [jnp.full((LANES,), i, jnp.int32)])
      x0, x1, wx0, wx1 = axis_prep(splat(oall_v, ray * 3) +
                                   splat(dall_v, ray * 3) * t)
      y0, y1, wy0, wy1 = axis_prep(splat(oall_v, ray * 3 + 1) +
                                   splat(dall_v, ray * 3 + 1) * t)
      z0, z1, wz0, wz1 = axis_prep(splat(oall_v, ray * 3 + 2) +
                                   splat(dall_v, ray * 3 + 2) * t)
      c = 0
      for zc, wz in ((z0, wz0), (z1, wz1)):
        for yc, wy in ((y0, wy0), (y1, wy1)):
          for xc, wx in ((x0, wx0), (x1, wx1)):
            idx_v[s, c, sl] = row_base + (zc * H + yc) * W + xc
            w_v[s, c, sl] = wz * wy * wx
            c += 1
      return c2
    lax.fori_loop(0, NGRP, build, 0)

    for c in range(NCORN):
      pltpu.async_copy(tab.at[idx_v.at[s, c]],
                       corner_v.at[pl.ds((s * NCORN + c) * CHK, CHK)],
                       sem_g.at[s])
    nbase = wid * PPW + jnp.minimum(k + 1, NCHUNK - 1) * CHK
    pltpu.async_copy(tlen.at[pl.ds(nbase, CHK)], len_v.at[1 - s, 0],
                     sem_p.at[1 - s])

  def stage_b(k, s):
    for c in range(NCORN):
      pltpu.make_async_copy(tab.at[idx_v.at[s, c]],
                            corner_v.at[pl.ds((s * NCORN + c) * CHK, CHK)],
                            sem_g.at[s]).wait()
    pltpu.make_async_copy(dens_v.at[s], dummy2.at[pl.ds(0, CHK)],
                          sem_o.at[s]).wait()
    pltpu.make_async_copy(feat_v.at[pl.ds(s * CHK * 8, CHK * 8)],
                          dummy2.at[pl.ds(0, CHK * 8)], sem_o.at[s]).wait()

    def interp(g, c2):
      sl = pl.ds(g * LANES, LANES)
      p_vec = g * LANES + lane
      accs = [jnp.zeros((LANES,), jnp.float32) for _ in range(9)]
      for c in range(NCORN):
        wv = w_v[s, c, sl]
        for w in range(5):
          w_vec = jnp.full((LANES,), w, jnp.int32)
          word = plsc.load_gather(corner_v,
                                  [p_vec + (s * NCORN + c) * CHK, w_vec])
          packed = plsc.bitcast(word, jnp.bfloat16)
          va, vb = plsc.unpack(packed, format=plsc.PackFormat.INTERLEAVED)
          accs[2 * w] = accs[2 * w] + va.astype(jnp.float32) * wv
          if w < 4:
            accs[2 * w + 1] = accs[2 * w + 1] + vb.astype(jnp.float32) * wv
      dens_v[s, sl] = accs[0]
      for ch in range(1, 9):
        plsc.store_scatter(feat_v,
                           [s * CHK * 8 + (g * LANES + lane) * 8 + (ch - 1)],
                           accs[ch])
      return c2
    lax.fori_loop(0, NGRP, interp, 0)
    base = wid * PPW + k * CHK
    pltpu.async_copy(dens_v.at[s], dens_out.at[pl.ds(base, CHK)], sem_o.at[s])
    pltpu.async_copy(feat_v.at[pl.ds(s * CHK * 8, CHK * 8)],
                     feat_out.at[pl.ds(base * 8, CHK * 8)], sem_o.at[s])

  for s in (0, 1):
    pltpu.async_copy(dens_v.at[s], dummy2.at[pl.ds(0, CHK)], sem_o.at[s])
    pltpu.async_copy(feat_v.at[pl.ds(s * CHK * 8, CHK * 8)],
                     dummy2.at[pl.ds(0, CHK * 8)], sem_o.at[s])

  stage_a(0, 0)
  stage_a(1, 1)
  stage_b(0, 0)

  def sample_pair(ko, carry):
    stage_a(2 * ko + 2, 0)
    stage_b(2 * ko + 1, 1)
    stage_a(2 * ko + 3, 1)
    stage_b(2 * ko + 2, 0)
    return carry
  lax.fori_loop(0, (NCHUNK - 2) // 2, sample_pair, 0)
  stage_b(NCHUNK - 1, 1)

  pltpu.make_async_copy(tlen.at[pl.ds(pbase0, CHK)],
                        len_v.at[0, 0], sem_p.at[0]).wait()
  for s in (0, 1):
    pltpu.make_async_copy(dens_v.at[s], dummy2.at[pl.ds(0, CHK)],
                          sem_o.at[s]).wait()
    pltpu.make_async_copy(feat_v.at[pl.ds(s * CHK * 8, CHK * 8)],
                          dummy2.at[pl.ds(0, CHK * 8)], sem_o.at[s]).wait()


def kernel(origins, directions, lengths, densities, features, world2local):
  ones = jnp.ones(origins.shape[:-1] + (1,), dtype=origins.dtype)
  o_h = jnp.concatenate([origins, ones], axis=-1)
  o_loc = jnp.einsum('bnk,bkj->bnj', o_h, world2local)
  o_loc = o_loc[..., :3] / o_loc[..., 3:4]
  d_loc = jnp.einsum('bnk,bkj->bnj', directions, world2local[:, :3, :3])

  mesh = plsc.VectorSubcoreMesh(core_axis_name="c", subcore_axis_name="s")
  run = pl.kernel(
      _body,
      out_type=(jax.ShapeDtypeStruct((N,), jnp.float32),
                jax.ShapeDtypeStruct((N * 8,), jnp.float32),
                jax.ShapeDtypeStruct((B * DHW, 8), jnp.int32),
                jax.ShapeDtypeStruct((TB, 8), jnp.int32),
                jax.ShapeDtypeStruct((CHK * 8,), jnp.float32)),
      mesh=mesh,
      scratch_types=(
          pltpu.VMEM((2, 9, TB), jnp.float32),
          pltpu.VMEM((2 * TB, 8), jnp.int32),
          pltpu.VMEM((N // P // NW * 3,), jnp.float32),
          pltpu.VMEM((N // P // NW * 3,), jnp.float32),
          pltpu.VMEM((2, 1, CHK), jnp.float32),
          pltpu.VMEM((2, NCORN, CHK), jnp.int32),
          pltpu.VMEM((2, NCORN, CHK), jnp.float32),
          pltpu.VMEM((2 * NCORN * CHK, 8), jnp.int32),
          pltpu.VMEM((2, CHK), jnp.float32),
          pltpu.VMEM((2 * CHK * 8,), jnp.float32),
          pltpu.SemaphoreType.DMA((2,)),
          pltpu.SemaphoreType.DMA((2,)),
          pltpu.SemaphoreType.DMA((2,)),
          pltpu.SemaphoreType.DMA((2,)),
          pltpu.SemaphoreType.DMA((2,)),
      ),
      compiler_params=pltpu.CompilerParams(needs_layout_passes=False,
                                           use_tc_tiling_on_sc=False),
  )
  dens_flat, feat_flat, _, _, _ = run(densities.reshape(-1),
                                      features.reshape(-1),
                                      o_loc.reshape(-1), d_loc.reshape(-1),
                                      lengths.reshape(-1))
  return (dens_flat.reshape(B, NR, P, 1), feat_flat.reshape(B, NR, P, 8))


# speedup vs baseline: 14.0640x; 1.0036x over previous
import jax
import jax.numpy as jnp
from jax import lax
from jax.experimental import pallas as pl
from jax.experimental.pallas import tpu as pltpu
from jax.experimental.pallas import tpu_sc as plsc

B, NR, P = 4, 2048, 64
D = H = W = 128
DHW = D * H * W
N = B * NR * P
NC, NS = 2, 16
NW = NC * NS
PPW = N // NW
CHK = 128
NCHUNK = PPW // CHK
NCORN = 8
LANES = 16
NGRP = CHK // LANES
VPW = B * DHW // NW
TB = 2048
NBCHUNK = VPW // TB


def _body(dens, feat, oall, dall, tlen, dens_out, feat_out, tab, dummy, dummy2,
          strip_v, rows_v, oall_v, dall_v, len_v, idx_v, w_v, corner_v,
          dens_v, feat_v, sem_s, sem_w, sem_p, sem_g, sem_o):
  wid = lax.axis_index("c") * NS + lax.axis_index("s")
  lane = lax.broadcasted_iota(jnp.int32, (LANES,), 0)

  pbase0 = wid * PPW
  RPW = PPW // P
  pltpu.sync_copy(oall.at[pl.ds(wid * RPW * 3, RPW * 3)], oall_v)
  pltpu.sync_copy(dall.at[pl.ds(wid * RPW * 3, RPW * 3)], dall_v)
  pltpu.async_copy(tlen.at[pl.ds(pbase0, CHK)], len_v.at[0, 0], sem_p.at[0])

  slab = wid * VPW
  bb = slab // DHW
  lv0 = slab - bb * DHW

  def strips_issue(k, s):
    lv = lv0 + k * TB
    pltpu.async_copy(dens.at[pl.ds(bb * DHW + lv, TB)],
                     strip_v.at[s, 0], sem_s.at[s])
    for ch in range(8):
      pltpu.async_copy(feat.at[pl.ds((bb * 8 + ch) * DHW + lv, TB)],
                       strip_v.at[s, ch + 1], sem_s.at[s])

  def strips_wait(s):
    for ch in range(9):
      pltpu.make_async_copy(dens.at[pl.ds(bb * DHW, TB)],
                            strip_v.at[s, ch], sem_s.at[s]).wait()

  def build_step(k, s):
    strips_wait(s)
    strips_issue(jnp.minimum(k + 1, NBCHUNK - 1), 1 - s)

    zero16 = jnp.zeros((LANES,), jnp.float32)

    def interleave(g, c2):
      sl = pl.ds(g * LANES, LANES)
      row_idx = s * TB + g * LANES + lane
      for w in range(5):
        va = strip_v[s, 2 * w, sl]
        vb = strip_v[s, 2 * w + 1, sl] if w < 4 else zero16
        packed = plsc.pack(va, vb, format=plsc.PackFormat.INTERLEAVED)
        word = plsc.bitcast(packed, jnp.int32)
        w_vec = jnp.full((LANES,), w, jnp.int32)
        plsc.store_scatter(rows_v, [row_idx, w_vec], word)
      return c2
    pltpu.make_async_copy(rows_v.at[pl.ds(s * TB, TB)],
                          dummy.at[pl.ds(0, TB)], sem_w.at[s]).wait()
    lax.fori_loop(0, TB // LANES, interleave, 0)
    pltpu.async_copy(rows_v.at[pl.ds(s * TB, TB)],
                     tab.at[pl.ds(slab + k * TB, TB)], sem_w.at[s])

  for s in (0, 1):
    pltpu.async_copy(rows_v.at[pl.ds(s * TB, TB)],
                     dummy.at[pl.ds(0, TB)], sem_w.at[s])
  strips_issue(0, 0)

  def build_pair(ko, carry):
    build_step(2 * ko, 0)
    build_step(2 * ko + 1, 1)
    return carry
  lax.fori_loop(0, NBCHUNK // 2, build_pair, 0)

  strips_wait(0)
  for s in (0, 1):
    pltpu.make_async_copy(rows_v.at[pl.ds(s * TB, TB)],
                          tab.at[pl.ds(slab, TB)], sem_w.at[s]).wait()
  plsc.subcore_barrier()

  def axis_prep(coord):
    f = (coord + 1.0) * ((D - 1) * 0.5)
    t = f.astype(jnp.int32)
    i0 = t - jnp.where(t.astype(jnp.float32) > f, 1, 0)
    w1 = f - i0.astype(jnp.float32)
    w0 = 1.0 - w1
    i1 = i0 + 1
    w0m = jnp.where((i0 >= 0) & (i0 < D), w0, 0.0)
    w1m = jnp.where((i1 >= 0) & (i1 < D), w1, 0.0)
    return jnp.clip(i0, 0, D - 1), jnp.clip(i1, 0, D - 1), w0m, w1m

  b = wid // (NW // B)
  row_base = b * DHW

  def stage_a(k, s):
    pltpu.make_async_copy(tlen.at[pl.ds(pbase0, CHK)],
                          len_v.at[s, 0], sem_p.at[s]).wait()

    def build(g, c2):
      sl = pl.ds(g * LANES, LANES)
      t = len_v[s, 0, sl]
      ray = k * (CHK // P) + g // (P // LANES)
      def splat(ref, i):
        return plsc.load_gather(ref, [jnp.full((LANES,), i, jnp.int32)])
      x0, x1, wx0, wx1 = axis_prep(splat(oall_v, ray * 3) +
                                   splat(dall_v, ray * 3) * t)
      y0, y1, wy0, wy1 = axis_prep(splat(oall_v, ray * 3 + 1) +
                                   splat(dall_v, ray * 3 + 1) * t)
      z0, z1, wz0, wz1 = axis_prep(splat(oall_v, ray * 3 + 2) +
                                   splat(dall_v, ray * 3 + 2) * t)
      c = 0
      for zc, wz in ((z0, wz0), (z1, wz1)):
        for yc, wy in ((y0, wy0), (y1, wy1)):
          for xc, wx in ((x0, wx0), (x1, wx1)):
            idx_v[s, c, sl] = row_base + (zc * H + yc) * W + xc
            w_v[s, c, sl] = wz * wy * wx
            c += 1
      return c2
    lax.fori_loop(0, NGRP, build, 0)

    for c in range(NCORN):
      pltpu.async_copy(tab.at[idx_v.at[s, c]],
                       corner_v.at[pl.ds((s * NCORN + c) * CHK, CHK)],
                       sem_g.at[s])
    nbase = wid * PPW + jnp.minimum(k + 1, NCHUNK - 1) * CHK
    pltpu.async_copy(tlen.at[pl.ds(nbase, CHK)], len_v.at[1 - s, 0],
                     sem_p.at[1 - s])

  def stage_b(k, s):
    for c in range(NCORN):
      pltpu.make_async_copy(tab.at[idx_v.at[s, c]],
                            corner_v.at[pl.ds((s * NCORN + c) * CHK, CHK)],
                            sem_g.at[s]).wait()
    pltpu.make_async_copy(dens_v.at[s], dummy2.at[pl.ds(0, CHK)],
                          sem_o.at[s]).wait()
    pltpu.make_async_copy(feat_v.at[pl.ds(s * CHK * 8, CHK * 8)],
                          dummy2.at[pl.ds(0, CHK * 8)], sem_o.at[s]).wait()

    def interp(g, c2):
      sl = pl.ds(g * LANES, LANES)
      p_vec = g * LANES + lane
      accs = [jnp.zeros((LANES,), jnp.float32) for _ in range(9)]
      for c in range(NCORN):
        wv = w_v[s, c, sl]
        for w in range(5):
          w_vec = jnp.full((LANES,), w, jnp.int32)
          word = plsc.load_gather(corner_v,
                                  [p_vec + (s * NCORN + c) * CHK, w_vec])
          packed = plsc.bitcast(word, jnp.bfloat16)
          va, vb = plsc.unpack(packed, format=plsc.PackFormat.INTERLEAVED)
          accs[2 * w] = accs[2 * w] + va.astype(jnp.float32) * wv
          if w < 4:
            accs[2 * w + 1] = accs[2 * w + 1] + vb.astype(jnp.float32) * wv
      dens_v[s, sl] = accs[0]
      for ch in range(1, 9):
        plsc.store_scatter(feat_v,
                           [s * CHK * 8 + (g * LANES + lane) * 8 + (ch - 1)],
                           accs[ch])
      return c2
    lax.fori_loop(0, NGRP, interp, 0)
    base = wid * PPW + k * CHK
    pltpu.async_copy(dens_v.at[s], dens_out.at[pl.ds(base, CHK)], sem_o.at[s])
    pltpu.async_copy(feat_v.at[pl.ds(s * CHK * 8, CHK * 8)],
                     feat_out.at[pl.ds(base * 8, CHK * 8)], sem_o.at[s])

  for s in (0, 1):
    pltpu.async_copy(dens_v.at[s], dummy2.at[pl.ds(0, CHK)], sem_o.at[s])
    pltpu.async_copy(feat_v.at[pl.ds(s * CHK * 8, CHK * 8)],
                     dummy2.at[pl.ds(0, CHK * 8)], sem_o.at[s])

  stage_a(0, 0)
  stage_a(1, 1)
  stage_b(0, 0)

  def sample_pair(ko, carry):
    stage_a(2 * ko + 2, 0)
    stage_b(2 * ko + 1, 1)
    stage_a(2 * ko + 3, 1)
    stage_b(2 * ko + 2, 0)
    return carry
  lax.fori_loop(0, (NCHUNK - 2) // 2, sample_pair, 0)
  stage_b(NCHUNK - 1, 1)

  pltpu.make_async_copy(tlen.at[pl.ds(pbase0, CHK)],
                        len_v.at[0, 0], sem_p.at[0]).wait()
  for s in (0, 1):
    pltpu.make_async_copy(dens_v.at[s], dummy2.at[pl.ds(0, CHK)],
                          sem_o.at[s]).wait()
    pltpu.make_async_copy(feat_v.at[pl.ds(s * CHK * 8, CHK * 8)],
                          dummy2.at[pl.ds(0, CHK * 8)], sem_o.at[s]).wait()


def kernel(origins, directions, lengths, densities, features, world2local):
  ones = jnp.ones(origins.shape[:-1] + (1,), dtype=origins.dtype)
  o_h = jnp.concatenate([origins, ones], axis=-1)
  o_loc = jnp.einsum('bnk,bkj->bnj', o_h, world2local)
  o_loc = o_loc[..., :3] / o_loc[..., 3:4]
  d_loc = jnp.einsum('bnk,bkj->bnj', directions, world2local[:, :3, :3])

  mesh = plsc.VectorSubcoreMesh(core_axis_name="c", subcore_axis_name="s")
  run = pl.kernel(
      _body,
      out_type=(jax.ShapeDtypeStruct((N,), jnp.float32),
                jax.ShapeDtypeStruct((N * 8,), jnp.float32),
                jax.ShapeDtypeStruct((B * DHW, 8), jnp.int32),
                jax.ShapeDtypeStruct((TB, 8), jnp.int32),
                jax.ShapeDtypeStruct((CHK * 8,), jnp.float32)),
      mesh=mesh,
      scratch_types=(
          pltpu.VMEM((2, 9, TB), jnp.float32),
          pltpu.VMEM((2 * TB, 8), jnp.int32),
          pltpu.VMEM((N // P // NW * 3,), jnp.float32),
          pltpu.VMEM((N // P // NW * 3,), jnp.float32),
          pltpu.VMEM((2, 1, CHK), jnp.float32),
          pltpu.VMEM((2, NCORN, CHK), jnp.int32),
          pltpu.VMEM((2, NCORN, CHK), jnp.float32),
          pltpu.VMEM((2 * NCORN * CHK, 8), jnp.int32),
          pltpu.VMEM((2, CHK), jnp.float32),
          pltpu.VMEM((2 * CHK * 8,), jnp.float32),
          pltpu.SemaphoreType.DMA((2,)),
          pltpu.SemaphoreType.DMA((2,)),
          pltpu.SemaphoreType.DMA((2,)),
          pltpu.SemaphoreType.DMA((2,)),
          pltpu.SemaphoreType.DMA((2,)),
      ),
      compiler_params=pltpu.CompilerParams(needs_layout_passes=False,
                                           use_tc_tiling_on_sc=False),
  )
  dens_flat, feat_flat, _, _, _ = run(densities.reshape(-1),
                                      features.reshape(-1),
                                      o_loc.reshape(-1), d_loc.reshape(-1),
                                      lengths.reshape(-1))
  return (dens_flat.reshape(B, NR, P, 1), feat_flat.reshape(B, NR, P, 8))
